# baseline (speedup 1.0000x reference)
import functools

import jax
import jax.numpy as jnp
from jax import lax
from jax.experimental import pallas as pl
from jax.experimental.pallas import tpu as pltpu
from jax.experimental.pallas import tpu_sc as plsc

N = 10000
DH = 128
NC = 2
NS = 16
NODES_PER_TILE = N // NS
ZROWS = 125

_MESH = plsc.VectorSubcoreMesh(
    core_axis_name="c", subcore_axis_name="s", num_cores=NC, num_subcores=NS)


def _deg_body(dst_hbm, out_hbm, dst_v, deg_v):
    c = lax.axis_index("c")
    s = lax.axis_index("s")
    w = c * NS + s
    pltpu.sync_copy(dst_hbm.at[w], dst_v)
    zeros = jnp.zeros((16,), jnp.float32)
    ones = jnp.full((16,), 1.0, jnp.float32)

    def zero_body(i, carry):
        deg_v[pl.ds(i * 16, 16)] = zeros
        return carry
    lax.fori_loop(0, N // 16, zero_body, 0)

    ept = dst_v.shape[0]

    def count_body(i, carry):
        idx = dst_v[pl.ds(i * 16, 16)]
        plsc.addupdate_scatter(deg_v, [idx], ones)
        return carry
    lax.fori_loop(0, ept // 16, count_body, 0)
    pltpu.sync_copy(deg_v, out_hbm.at[w])


def _deg_partials(dst_r):
    ept = dst_r.shape[1]
    return pl.kernel(
        _deg_body,
        out_type=jax.ShapeDtypeStruct((NC * NS, N), jnp.float32),
        mesh=_MESH,
        compiler_params=pltpu.CompilerParams(needs_layout_passes=False),
        scratch_types=[
            pltpu.VMEM((ept,), jnp.int32),
            pltpu.VMEM((N,), jnp.float32),
        ],
    )(dst_r)


SB = 8
NSLOT = 3


def _scatter_body(table_hbm, src_hbm, dst_hbm, zeros_hbm, out_hbm,
                  src_c, dst_c, gbuf, acc,
                  gsem0, gsem1, gsem2, ssem0, ssem1, ssem2, isem0, isem1):
    c = lax.axis_index("c")
    s = lax.axis_index("s")
    w = c * NS + s
    nb = src_hbm.shape[1]
    nq = nb // SB

    @pl.when(s < N // 1000)
    def _():
        off = pl.multiple_of(s * 1000, 8)
        pltpu.sync_copy(zeros_hbm, acc.at[pl.ds(off, 1000)])

    gsems = (gsem0, gsem1, gsem2)
    ssems = (ssem0, ssem1, ssem2)
    isems = (isem0, isem1)

    def idx_start(q):
        j = q % 2
        off = pl.multiple_of(q * SB, 8)
        pltpu.async_copy(src_hbm.at[w].at[pl.ds(off, SB)], src_c.at[j], isems[j])
        pltpu.async_copy(dst_hbm.at[s].at[pl.ds(off, SB)], dst_c.at[j], isems[j])

    def idx_wait(q):
        j = q % 2
        off = pl.multiple_of(q * SB, 8)
        pltpu.make_async_copy(src_hbm.at[w].at[pl.ds(off, SB)], src_c.at[j],
                              isems[j]).wait()
        pltpu.make_async_copy(dst_hbm.at[s].at[pl.ds(off, SB)], dst_c.at[j],
                              isems[j]).wait()

    def gsrc(bb):
        return src_c.at[(bb // SB) % 2].at[bb % SB]

    def gdst(bb):
        return dst_c.at[(bb // SB) % 2].at[bb % SB]

    def gstart(bb):
        k = bb % NSLOT
        pltpu.async_copy(table_hbm.at[gsrc(bb)], gbuf.at[k], gsems[k])

    def gwait(bb):
        k = bb % NSLOT
        pltpu.make_async_copy(table_hbm.at[gsrc(bb)], gbuf.at[k],
                              gsems[k]).wait()

    def sstart(bb):
        k = bb % NSLOT
        pltpu.async_copy(gbuf.at[k], acc.at[gdst(bb)], ssems[k], add=True)

    def swait(bb):
        k = bb % NSLOT
        pltpu.make_async_copy(gbuf.at[k], acc.at[gdst(bb)], ssems[k]).wait()

    idx_start(0)
    idx_wait(0)
    gstart(0)
    gstart(1)
    plsc.subcore_barrier()

    for bb in range(nb):
        q = bb // SB
        if bb % SB == 0 and q + 1 < nq:
            idx_start(q + 1)
        if bb % SB == SB - 2 and q + 1 < nq:
            idx_wait(q + 1)
        if bb >= 1:
            swait(bb - 1)
        if bb + 2 < nb:
            gstart(bb + 2)
        gwait(bb)
        sstart(bb)
    swait(nb - 1)

    plsc.subcore_barrier()

    @pl.when(s < N // 1000)
    def _():
        off = pl.multiple_of(s * 1000, 8)
        pltpu.sync_copy(acc.at[pl.ds(off, 1000)],
                        out_hbm.at[pl.ds(c * N + off, 1000)])


def _sc_scatter(table, src_r, dst_r, zeros):
    b = src_r.shape[2]
    return pl.kernel(
        _scatter_body,
        out_type=jax.ShapeDtypeStruct((NC * N, DH), jnp.float32),
        mesh=_MESH,
        scratch_types=[
            pltpu.VMEM((2, SB, b), jnp.int32),
            pltpu.VMEM((2, SB, b), jnp.int32),
            pltpu.VMEM((NSLOT, b, DH), jnp.float32),
            pltpu.VMEM_SHARED((N, DH), jnp.float32),
            pltpu.SemaphoreType.DMA,
            pltpu.SemaphoreType.DMA,
            pltpu.SemaphoreType.DMA,
            pltpu.SemaphoreType.DMA,
            pltpu.SemaphoreType.DMA,
            pltpu.SemaphoreType.DMA,
            pltpu.SemaphoreType.DMA,
            pltpu.SemaphoreType.DMA,
        ],
    )(table, src_r, dst_r, zeros)


_BLK = 1000


def _dinv_body(degp_ref, out_ref):
    deg = jnp.sum(degp_ref[...], axis=0, keepdims=True) + 1.0
    out_ref[...] = lax.rsqrt(deg)


def _tc_dinv(degp):
    return pl.pallas_call(
        _dinv_body,
        out_shape=jax.ShapeDtypeStruct((1, N), jnp.float32),
    )(degp)


def _enc_body(x_ref, dinv_ref, w1, b1, w2, b2, cw1h, out_ref, h_s):
    dinv = dinv_ref[0, 0, :]

    @pl.when(pl.program_id(1) == 0)
    def _():
        h = jnp.maximum(jnp.dot(x_ref[...], w1[...],
                                preferred_element_type=jnp.float32) + b1[...], 0.0)
        h_s[...] = jnp.dot(h, w2[...],
                           preferred_element_type=jnp.float32) + b2[...]
    hl = jnp.dot(h_s[...], cw1h[...], preferred_element_type=jnp.float32)
    out_ref[...] = dinv[:, None] * hl


def _tc_encode(x, dinv3, w1, b1, w2, b2, cw1):
    g = N // _BLK
    return pl.pallas_call(
        _enc_body,
        grid=(g, 2),
        in_specs=[
            pl.BlockSpec((_BLK, 128), lambda i, j: (i, 0)),
            pl.BlockSpec((1, 1, _BLK), lambda i, j: (i, 0, 0)),
            pl.BlockSpec((128, 256), lambda i, j: (0, 0)),
            pl.BlockSpec((1, 256), lambda i, j: (0, 0)),
            pl.BlockSpec((256, 256), lambda i, j: (0, 0)),
            pl.BlockSpec((1, 256), lambda i, j: (0, 0)),
            pl.BlockSpec((256, DH), lambda i, j: (0, j)),
        ],
        out_specs=pl.BlockSpec((_BLK, DH), lambda i, j: (j * (N // _BLK) + i, 0)),
        out_shape=jax.ShapeDtypeStruct((NC * N, DH), jnp.float32),
        scratch_shapes=[pltpu.VMEM((_BLK, 256), jnp.float32)],
    )(x, dinv3, w1, b1, w2, b2, cw1)


def _mid_body(acc_lo, acc_hi, tab_lo, tab_hi, dinv_ref, b_ref, wnh_ref, out_ref):
    dinv = dinv_ref[0, 0, :]
    acc = jnp.concatenate([acc_lo[...], acc_hi[...]], axis=1)
    hlp = jnp.concatenate([tab_lo[...], tab_hi[...]], axis=1)
    t = jnp.maximum(dinv[:, None] * (acc + hlp) + b_ref[...], 0.0)
    out_ref[...] = dinv[:, None] * jnp.dot(t, wnh_ref[...],
                                           preferred_element_type=jnp.float32)


def _tc_mid(accf, table, dinv3, b, wn):
    g = N // _BLK
    return pl.pallas_call(
        _mid_body,
        grid=(g, 2),
        in_specs=[
            pl.BlockSpec((_BLK, DH), lambda i, j: (i, 0)),
            pl.BlockSpec((_BLK, DH), lambda i, j: (g + i, 0)),
            pl.BlockSpec((_BLK, DH), lambda i, j: (i, 0)),
            pl.BlockSpec((_BLK, DH), lambda i, j: (g + i, 0)),
            pl.BlockSpec((1, 1, _BLK), lambda i, j: (i, 0, 0)),
            pl.BlockSpec((1, 256), lambda i, j: (0, 0)),
            pl.BlockSpec((256, DH), lambda i, j: (0, j)),
        ],
        out_specs=pl.BlockSpec((_BLK, DH), lambda i, j: (j * g + i, 0)),
        out_shape=jax.ShapeDtypeStruct((NC * N, DH), jnp.float32),
    )(accf, accf, table, table, dinv3, b, wn)


def _final_body(acc_lo, acc_hi, tab_lo, tab_hi, dinv_ref, b3_ref,
                pw1, pb1, pw2, pb2, dw1, db1, dw2, db2,
                ge_ref, p_ref, sum_s, max_s):
    i = pl.program_id(0)
    dinv = dinv_ref[0, 0, :]
    acc = jnp.concatenate([acc_lo[...], acc_hi[...]], axis=1)
    hlp = jnp.concatenate([tab_lo[...], tab_hi[...]], axis=1)
    h3 = jnp.maximum(dinv[:, None] * (acc + hlp) + b3_ref[...], 0.0)
    bsum = jnp.sum(h3, axis=0, keepdims=True)
    bmax = jnp.max(h3, axis=0, keepdims=True)

    @pl.when(i == 0)
    def _():
        sum_s[...] = bsum
        max_s[...] = bmax

    @pl.when(i > 0)
    def _():
        sum_s[...] = sum_s[...] + bsum
        max_s[...] = jnp.maximum(max_s[...], bmax)

    @pl.when(i == N // _BLK - 1)
    def _():
        g = jnp.concatenate([sum_s[...] * (1.0 / N), max_s[...]], axis=1)
        ge = jnp.maximum(jnp.dot(g, pw1[...],
                                 preferred_element_type=jnp.float32)
                         + pb1[...], 0.0)
        ge = jnp.dot(ge, pw2[...], preferred_element_type=jnp.float32) + pb2[...]
        v = jnp.maximum(jnp.dot(ge, dw1[...],
                                preferred_element_type=jnp.float32)
                        + db1[...], 0.0)
        v = jnp.dot(v, dw2[...], preferred_element_type=jnp.float32) + db2[...]
        ge_ref[...] = ge
        p_ref[...] = jax.nn.sigmoid(jnp.sum(v * v)).reshape(1, 1)


def _tc_final(accf, table, dinv3, b3, pw1, pb1, pw2, pb2, dw1, db1, dw2, db2):
    g = N // _BLK
    hs = [pl.BlockSpec((_BLK, DH), lambda i: (i, 0)),
          pl.BlockSpec((_BLK, DH), lambda i: (g + i, 0))]
    return pl.pallas_call(
        _final_body,
        grid=(g,),
        in_specs=hs + hs + [pl.BlockSpec((1, 1, _BLK), lambda i: (i, 0, 0))]
                 + [pl.BlockSpec(x.shape, lambda i, nd=x.ndim: (0,) * nd)
                    for x in (b3, pw1, pb1, pw2, pb2, dw1, db1, dw2, db2)],
        out_specs=(pl.BlockSpec((1, 128), lambda i: (0, 0)),
                   pl.BlockSpec((1, 1), lambda i: (0, 0))),
        out_shape=(
            jax.ShapeDtypeStruct((1, 128), jnp.float32),
            jax.ShapeDtypeStruct((1, 1), jnp.float32),
        ),
        scratch_shapes=[pltpu.VMEM((1, 256), jnp.float32),
                        pltpu.VMEM((1, 256), jnp.float32)],
    )(accf, accf, table, table, dinv3, b3, pw1, pb1, pw2, pb2, dw1, db1, dw2, db2)


def kernel(x, edge_index, enc_W1, enc_b1, enc_W2, enc_b2,
           conv_W1, conv_b1, conv_W2, conv_b2, conv_W3, conv_b3,
           proj_W1, proj_b1, proj_W2, proj_b2,
           dec_W1, dec_b1, dec_W2, dec_b2):
    e = edge_index.shape[1]
    src = edge_index[0].astype(jnp.int32)
    dst = edge_index[1].astype(jnp.int32)

    ept_deg = e // (NC * NS)
    dst_deg = dst.reshape(NC * NS, ept_deg)
    degp = _deg_partials(dst_deg)

    b = 100
    nb = e // (NS * b)
    zeros = jnp.zeros((1000, DH), jnp.float32)
    src_t = src.reshape(NS, nb, b)
    src_r = jnp.concatenate([src_t, src_t + N], axis=0).reshape(NC * NS, nb, b)
    dst_r = dst.reshape(NS, nb, b)

    dinv = _tc_dinv(degp)
    dinv3 = dinv.reshape(N // _BLK, 1, _BLK)

    r2 = lambda v: v.reshape(1, -1)
    table = _tc_encode(x, dinv3, enc_W1, r2(enc_b1), enc_W2, r2(enc_b2), conv_W1)

    for (bb, wn) in ((conv_b1, conv_W2), (conv_b2, conv_W3)):
        accf = _sc_scatter(table, src_r, dst_r, zeros)
        table = _tc_mid(accf, table, dinv3, r2(bb), wn)

    accf = _sc_scatter(table, src_r, dst_r, zeros)

    ge, pv = _tc_final(accf, table, dinv3, r2(conv_b3),
                       proj_W1, r2(proj_b1), proj_W2, r2(proj_b2),
                       dec_W1, r2(dec_b1), dec_W2, r2(dec_b2))
    probs = jnp.broadcast_to(pv.reshape(()), (e,))
    return ge, probs


# speedup vs baseline: 1.0045x; 1.0045x over previous
import jax
import jax.numpy as jnp
from jax import lax
from jax.experimental import pallas as pl
from jax.experimental.pallas import tpu as pltpu
from jax.experimental.pallas import tpu_sc as plsc

N = 10000
DH = 128
NC = 2
NS = 16
NODES_PER_TILE = N // NS
ZROWS = 125

_MESH = plsc.VectorSubcoreMesh(
    core_axis_name="c", subcore_axis_name="s", num_cores=NC, num_subcores=NS)


def _deg_body(dst_hbm, out_hbm, dst_v, deg_v):
    c = lax.axis_index("c")
    s = lax.axis_index("s")
    w = c * NS + s
    pltpu.sync_copy(dst_hbm.at[w], dst_v)
    zeros = jnp.zeros((16,), jnp.float32)
    ones = jnp.full((16,), 1.0, jnp.float32)

    def zero_body(i, carry):
        deg_v[pl.ds(i * 16, 16)] = zeros
        return carry
    lax.fori_loop(0, N // 16, zero_body, 0)

    ept = dst_v.shape[0]

    def count_body(i, carry):
        idx = dst_v[pl.ds(i * 16, 16)]
        plsc.addupdate_scatter(deg_v, [idx], ones)
        return carry
    lax.fori_loop(0, ept // 16, count_body, 0)
    pltpu.sync_copy(deg_v, out_hbm.at[w])


def _deg_partials(dst_r):
    ept = dst_r.shape[1]
    return pl.kernel(
        _deg_body,
        out_type=jax.ShapeDtypeStruct((NC * NS, N), jnp.float32),
        mesh=_MESH,
        compiler_params=pltpu.CompilerParams(needs_layout_passes=False),
        scratch_types=[
            pltpu.VMEM((ept,), jnp.int32),
            pltpu.VMEM((N,), jnp.float32),
        ],
    )(dst_r)


SB = 8
NSLOT = 3


def _scatter_body(table_hbm, src_hbm, dst_hbm, zeros_hbm, out_hbm,
                  src_c, dst_c, gbuf, acc,
                  gsem0, gsem1, gsem2, ssem0, ssem1, ssem2, isem0, isem1):
    c = lax.axis_index("c")
    s = lax.axis_index("s")
    w = c * NS + s
    nb = src_hbm.shape[1]
    nq = nb // SB

    @pl.when(s < N // 1000)
    def _():
        off = pl.multiple_of(s * 1000, 8)
        pltpu.sync_copy(zeros_hbm, acc.at[pl.ds(off, 1000)])

    gsems = (gsem0, gsem1, gsem2)
    ssems = (ssem0, ssem1, ssem2)
    isems = (isem0, isem1)

    def idx_start(q):
        j = q % 2
        off = pl.multiple_of(q * SB, 8)
        pltpu.async_copy(src_hbm.at[w].at[pl.ds(off, SB)], src_c.at[j], isems[j])
        pltpu.async_copy(dst_hbm.at[s].at[pl.ds(off, SB)], dst_c.at[j], isems[j])

    def idx_wait(q):
        j = q % 2
        off = pl.multiple_of(q * SB, 8)
        pltpu.make_async_copy(src_hbm.at[w].at[pl.ds(off, SB)], src_c.at[j],
                              isems[j]).wait()
        pltpu.make_async_copy(dst_hbm.at[s].at[pl.ds(off, SB)], dst_c.at[j],
                              isems[j]).wait()

    def gsrc(bb):
        return src_c.at[(bb // SB) % 2].at[bb % SB]

    def gdst(bb):
        return dst_c.at[(bb // SB) % 2].at[bb % SB]

    def gstart(bb):
        k = bb % NSLOT
        pltpu.async_copy(table_hbm.at[gsrc(bb)], gbuf.at[k], gsems[k])

    def gwait(bb):
        k = bb % NSLOT
        pltpu.make_async_copy(table_hbm.at[gsrc(bb)], gbuf.at[k],
                              gsems[k]).wait()

    def sstart(bb):
        k = bb % NSLOT
        pltpu.async_copy(gbuf.at[k], acc.at[gdst(bb)], ssems[k], add=True)

    def swait(bb):
        k = bb % NSLOT
        pltpu.make_async_copy(gbuf.at[k], acc.at[gdst(bb)], ssems[k]).wait()

    idx_start(0)
    idx_wait(0)
    gstart(0)
    gstart(1)
    plsc.subcore_barrier()

    for bb in range(nb):
        q = bb // SB
        if bb % SB == 0 and q + 1 < nq:
            idx_start(q + 1)
        if bb % SB == SB - 2 and q + 1 < nq:
            idx_wait(q + 1)
        if bb >= 1:
            swait(bb - 1)
        if bb + 2 < nb:
            gstart(bb + 2)
        gwait(bb)
        sstart(bb)
    swait(nb - 1)

    plsc.subcore_barrier()

    @pl.when(s < N // 1000)
    def _():
        off = pl.multiple_of(s * 1000, 8)
        pltpu.sync_copy(acc.at[pl.ds(off, 1000)],
                        out_hbm.at[pl.ds(c * N + off, 1000)])


def _sc_scatter(table, src_r, dst_r, zeros):
    b = src_r.shape[2]
    return pl.kernel(
        _scatter_body,
        out_type=jax.ShapeDtypeStruct((NC * N, DH), jnp.float32),
        mesh=_MESH,
        scratch_types=[
            pltpu.VMEM((2, SB, b), jnp.int32),
            pltpu.VMEM((2, SB, b), jnp.int32),
            pltpu.VMEM((NSLOT, b, DH), jnp.float32),
            pltpu.VMEM_SHARED((N, DH), jnp.float32),
            pltpu.SemaphoreType.DMA,
            pltpu.SemaphoreType.DMA,
            pltpu.SemaphoreType.DMA,
            pltpu.SemaphoreType.DMA,
            pltpu.SemaphoreType.DMA,
            pltpu.SemaphoreType.DMA,
            pltpu.SemaphoreType.DMA,
            pltpu.SemaphoreType.DMA,
        ],
    )(table, src_r, dst_r, zeros)


_BLK = 1000


def _dinv_body(degp_ref, out_ref):
    deg = jnp.sum(degp_ref[...], axis=0, keepdims=True) + 1.0
    out_ref[...] = lax.rsqrt(deg)


def _tc_dinv(degp):
    return pl.pallas_call(
        _dinv_body,
        out_shape=jax.ShapeDtypeStruct((1, N), jnp.float32),
    )(degp)


def _enc_body(x_ref, dinv_ref, w1, b1, w2, b2, cw1h, out_ref):
    dinv = dinv_ref[0, 0, :]
    h = jnp.maximum(jnp.dot(x_ref[...], w1[...],
                            preferred_element_type=jnp.float32) + b1[...], 0.0)
    h = jnp.dot(h, w2[...], preferred_element_type=jnp.float32) + b2[...]
    hl = jnp.dot(h, cw1h[...], preferred_element_type=jnp.float32)
    out_ref[...] = dinv[:, None] * hl


def _tc_encode(x, dinv3, w1, b1, w2, b2, cw1):
    g = N // _BLK
    return pl.pallas_call(
        _enc_body,
        grid=(g, 2),
        in_specs=[
            pl.BlockSpec((_BLK, 128), lambda i, j: (i, 0)),
            pl.BlockSpec((1, 1, _BLK), lambda i, j: (i, 0, 0)),
            pl.BlockSpec((128, 256), lambda i, j: (0, 0)),
            pl.BlockSpec((1, 256), lambda i, j: (0, 0)),
            pl.BlockSpec((256, 256), lambda i, j: (0, 0)),
            pl.BlockSpec((1, 256), lambda i, j: (0, 0)),
            pl.BlockSpec((256, DH), lambda i, j: (0, j)),
        ],
        out_specs=pl.BlockSpec((_BLK, DH), lambda i, j: (j * (N // _BLK) + i, 0)),
        out_shape=jax.ShapeDtypeStruct((NC * N, DH), jnp.float32),
    )(x, dinv3, w1, b1, w2, b2, cw1)


def _mid_body(acc_lo, acc_hi, tab_lo, tab_hi, dinv_ref, b_ref, wnh_ref, out_ref):
    dinv = dinv_ref[0, 0, :]
    acc = jnp.concatenate([acc_lo[...], acc_hi[...]], axis=1)
    hlp = jnp.concatenate([tab_lo[...], tab_hi[...]], axis=1)
    t = jnp.maximum(dinv[:, None] * (acc + hlp) + b_ref[...], 0.0)
    out_ref[...] = dinv[:, None] * jnp.dot(t, wnh_ref[...],
                                           preferred_element_type=jnp.float32)


def _tc_mid(accf, table, dinv3, b, wn):
    g = N // _BLK
    return pl.pallas_call(
        _mid_body,
        grid=(g, 2),
        in_specs=[
            pl.BlockSpec((_BLK, DH), lambda i, j: (i, 0)),
            pl.BlockSpec((_BLK, DH), lambda i, j: (g + i, 0)),
            pl.BlockSpec((_BLK, DH), lambda i, j: (i, 0)),
            pl.BlockSpec((_BLK, DH), lambda i, j: (g + i, 0)),
            pl.BlockSpec((1, 1, _BLK), lambda i, j: (i, 0, 0)),
            pl.BlockSpec((1, 256), lambda i, j: (0, 0)),
            pl.BlockSpec((256, DH), lambda i, j: (0, j)),
        ],
        out_specs=pl.BlockSpec((_BLK, DH), lambda i, j: (j * g + i, 0)),
        out_shape=jax.ShapeDtypeStruct((NC * N, DH), jnp.float32),
    )(accf, accf, table, table, dinv3, b, wn)


def _final_body(acc_lo, acc_hi, tab_lo, tab_hi, dinv_ref, b3_ref,
                pw1, pb1, pw2, pb2, dw1, db1, dw2, db2,
                ge_ref, p_ref, sum_s, max_s):
    i = pl.program_id(0)
    dinv = dinv_ref[0, 0, :]
    acc = jnp.concatenate([acc_lo[...], acc_hi[...]], axis=1)
    hlp = jnp.concatenate([tab_lo[...], tab_hi[...]], axis=1)
    h3 = jnp.maximum(dinv[:, None] * (acc + hlp) + b3_ref[...], 0.0)
    bsum = jnp.sum(h3, axis=0, keepdims=True)
    bmax = jnp.max(h3, axis=0, keepdims=True)

    @pl.when(i == 0)
    def _():
        sum_s[...] = bsum
        max_s[...] = bmax

    @pl.when(i > 0)
    def _():
        sum_s[...] = sum_s[...] + bsum
        max_s[...] = jnp.maximum(max_s[...], bmax)

    @pl.when(i == N // _BLK - 1)
    def _():
        g = jnp.concatenate([sum_s[...] * (1.0 / N), max_s[...]], axis=1)
        ge = jnp.maximum(jnp.dot(g, pw1[...],
                                 preferred_element_type=jnp.float32)
                         + pb1[...], 0.0)
        ge = jnp.dot(ge, pw2[...], preferred_element_type=jnp.float32) + pb2[...]
        v = jnp.maximum(jnp.dot(ge, dw1[...],
                                preferred_element_type=jnp.float32)
                        + db1[...], 0.0)
        v = jnp.dot(v, dw2[...], preferred_element_type=jnp.float32) + db2[...]
        ge_ref[...] = ge
        p_ref[...] = jax.nn.sigmoid(jnp.sum(v * v)).reshape(1, 1)


def _tc_final(accf, table, dinv3, b3, pw1, pb1, pw2, pb2, dw1, db1, dw2, db2):
    g = N // _BLK
    hs = [pl.BlockSpec((_BLK, DH), lambda i: (i, 0)),
          pl.BlockSpec((_BLK, DH), lambda i: (g + i, 0))]
    return pl.pallas_call(
        _final_body,
        grid=(g,),
        in_specs=hs + hs + [pl.BlockSpec((1, 1, _BLK), lambda i: (i, 0, 0))]
                 + [pl.BlockSpec(x.shape, lambda i, nd=x.ndim: (0,) * nd)
                    for x in (b3, pw1, pb1, pw2, pb2, dw1, db1, dw2, db2)],
        out_specs=(pl.BlockSpec((1, 128), lambda i: (0, 0)),
                   pl.BlockSpec((1, 1), lambda i: (0, 0))),
        out_shape=(
            jax.ShapeDtypeStruct((1, 128), jnp.float32),
            jax.ShapeDtypeStruct((1, 1), jnp.float32),
        ),
        scratch_shapes=[pltpu.VMEM((1, 256), jnp.float32),
                        pltpu.VMEM((1, 256), jnp.float32)],
    )(accf, accf, table, table, dinv3, b3, pw1, pb1, pw2, pb2, dw1, db1, dw2, db2)


def kernel(x, edge_index, enc_W1, enc_b1, enc_W2, enc_b2,
           conv_W1, conv_b1, conv_W2, conv_b2, conv_W3, conv_b3,
           proj_W1, proj_b1, proj_W2, proj_b2,
           dec_W1, dec_b1, dec_W2, dec_b2):
    e = edge_index.shape[1]
    src = edge_index[0].astype(jnp.int32)
    dst = edge_index[1].astype(jnp.int32)

    ept_deg = e // (NC * NS)
    dst_deg = dst.reshape(NC * NS, ept_deg)
    degp = _deg_partials(dst_deg)

    b = 100
    nb = e // (NS * b)
    zeros = jnp.zeros((1000, DH), jnp.float32)
    src_t = src.reshape(NS, nb, b)
    src_r = jnp.concatenate([src_t, src_t + N], axis=0).reshape(NC * NS, nb, b)
    dst_r = dst.reshape(NS, nb, b)

    dinv = _tc_dinv(degp)
    dinv3 = dinv.reshape(N // _BLK, 1, _BLK)

    r2 = lambda v: v.reshape(1, -1)
    table = _tc_encode(x, dinv3, enc_W1, r2(enc_b1), enc_W2, r2(enc_b2), conv_W1)

    for (bb, wn) in ((conv_b1, conv_W2), (conv_b2, conv_W3)):
        accf = _sc_scatter(table, src_r, dst_r, zeros)
        table = _tc_mid(accf, table, dinv3, r2(bb), wn)

    accf = _sc_scatter(table, src_r, dst_r, zeros)

    ge, pv = _tc_final(accf, table, dinv3, r2(conv_b3),
                       proj_W1, r2(proj_b1), proj_W2, r2(proj_b2),
                       dec_W1, r2(dec_b1), dec_W2, r2(dec_b2))
    probs = jnp.broadcast_to(pv.reshape(()), (e,))
    return ge, probs


# speedup vs baseline: 1.0053x; 1.0008x over previous
import jax
import jax.numpy as jnp
from jax import lax
from jax.experimental import pallas as pl
from jax.experimental.pallas import tpu as pltpu
from jax.experimental.pallas import tpu_sc as plsc

N = 10000
DH = 128
NC = 2
NS = 16
_MESH = plsc.VectorSubcoreMesh(
    core_axis_name="c", subcore_axis_name="s", num_cores=NC, num_subcores=NS)


def _deg_body(dst_hbm, out_hbm, dst_v, deg_v):
    c = lax.axis_index("c")
    s = lax.axis_index("s")
    w = c * NS + s
    pltpu.sync_copy(dst_hbm.at[w], dst_v)
    zeros = jnp.zeros((16,), jnp.float32)
    ones = jnp.full((16,), 1.0, jnp.float32)

    def zero_body(i, carry):
        deg_v[pl.ds(i * 16, 16)] = zeros
        return carry
    lax.fori_loop(0, N // 16, zero_body, 0)

    ept = dst_v.shape[0]

    def count_body(i, carry):
        idx = dst_v[pl.ds(i * 16, 16)]
        plsc.addupdate_scatter(deg_v, [idx], ones)
        return carry
    lax.fori_loop(0, ept // 16, count_body, 0)
    pltpu.sync_copy(deg_v, out_hbm.at[w])


def _deg_partials(dst_r):
    ept = dst_r.shape[1]
    return pl.kernel(
        _deg_body,
        out_type=jax.ShapeDtypeStruct((NC * NS, N), jnp.float32),
        mesh=_MESH,
        compiler_params=pltpu.CompilerParams(needs_layout_passes=False),
        scratch_types=[
            pltpu.VMEM((ept,), jnp.int32),
            pltpu.VMEM((N,), jnp.float32),
        ],
    )(dst_r)


SB = 8
NSLOT = 3


def _scatter_body(table_hbm, src_hbm, dst_hbm, zeros_hbm, out_hbm,
                  src_c, dst_c, gbuf, acc,
                  gsem0, gsem1, gsem2, ssem0, ssem1, ssem2, isem0, isem1):
    c = lax.axis_index("c")
    s = lax.axis_index("s")
    w = c * NS + s
    nb = src_hbm.shape[1]
    nq = nb // SB

    @pl.when(s < N // 1000)
    def _():
        off = pl.multiple_of(s * 1000, 8)
        pltpu.sync_copy(zeros_hbm, acc.at[pl.ds(off, 1000)])

    gsems = (gsem0, gsem1, gsem2)
    ssems = (ssem0, ssem1, ssem2)
    isems = (isem0, isem1)

    def idx_start(q):
        j = q % 2
        off = pl.multiple_of(q * SB, 8)
        pltpu.async_copy(src_hbm.at[w].at[pl.ds(off, SB)], src_c.at[j], isems[j])
        pltpu.async_copy(dst_hbm.at[s].at[pl.ds(off, SB)], dst_c.at[j], isems[j])

    def idx_wait(q):
        j = q % 2
        off = pl.multiple_of(q * SB, 8)
        pltpu.make_async_copy(src_hbm.at[w].at[pl.ds(off, SB)], src_c.at[j],
                              isems[j]).wait()
        pltpu.make_async_copy(dst_hbm.at[s].at[pl.ds(off, SB)], dst_c.at[j],
                              isems[j]).wait()

    def gsrc(bb):
        return src_c.at[(bb // SB) % 2].at[bb % SB]

    def gdst(bb):
        return dst_c.at[(bb // SB) % 2].at[bb % SB]

    def gstart(bb):
        k = bb % NSLOT
        pltpu.async_copy(table_hbm.at[gsrc(bb)], gbuf.at[k], gsems[k])

    def gwait(bb):
        k = bb % NSLOT
        pltpu.make_async_copy(table_hbm.at[gsrc(bb)], gbuf.at[k],
                              gsems[k]).wait()

    def sstart(bb):
        k = bb % NSLOT
        pltpu.async_copy(gbuf.at[k], acc.at[gdst(bb)], ssems[k], add=True)

    def swait(bb):
        k = bb % NSLOT
        pltpu.make_async_copy(gbuf.at[k], acc.at[gdst(bb)], ssems[k]).wait()

    idx_start(0)
    idx_wait(0)
    gstart(0)
    gstart(1)
    plsc.subcore_barrier()

    for bb in range(nb):
        q = bb // SB
        if bb % SB == 0 and q + 1 < nq:
            idx_start(q + 1)
        if bb % SB == SB - 2 and q + 1 < nq:
            idx_wait(q + 1)
        if bb >= 1:
            swait(bb - 1)
        if bb + 2 < nb:
            gstart(bb + 2)
        gwait(bb)
        sstart(bb)
    swait(nb - 1)

    plsc.subcore_barrier()

    @pl.when(s < N // 1000)
    def _():
        off = pl.multiple_of(s * 1000, 8)
        pltpu.sync_copy(acc.at[pl.ds(off, 1000)],
                        out_hbm.at[pl.ds(c * N + off, 1000)])


def _sc_scatter(table, src_r, dst_r, zeros):
    b = src_r.shape[2]
    return pl.kernel(
        _scatter_body,
        out_type=jax.ShapeDtypeStruct((NC * N, DH), jnp.float32),
        mesh=_MESH,
        scratch_types=[
            pltpu.VMEM((2, SB, b), jnp.int32),
            pltpu.VMEM((2, SB, b), jnp.int32),
            pltpu.VMEM((NSLOT, b, DH), jnp.float32),
            pltpu.VMEM_SHARED((N, DH), jnp.float32),
            pltpu.SemaphoreType.DMA,
            pltpu.SemaphoreType.DMA,
            pltpu.SemaphoreType.DMA,
            pltpu.SemaphoreType.DMA,
            pltpu.SemaphoreType.DMA,
            pltpu.SemaphoreType.DMA,
            pltpu.SemaphoreType.DMA,
            pltpu.SemaphoreType.DMA,
        ],
    )(table, src_r, dst_r, zeros)


_BLK = 1000


def _dinv_body(degp_ref, out_ref):
    deg = jnp.sum(degp_ref[...], axis=0, keepdims=True) + 1.0
    out_ref[...] = lax.rsqrt(deg)


def _tc_dinv(degp):
    return pl.pallas_call(
        _dinv_body,
        out_shape=jax.ShapeDtypeStruct((1, N), jnp.float32),
    )(degp)


def _enc_body(x_ref, dinv_ref, w1, b1, w2, b2, cw1h, out_ref):
    dinv = dinv_ref[0, 0, :]
    h = jnp.maximum(jnp.dot(x_ref[...], w1[...],
                            preferred_element_type=jnp.float32) + b1[...], 0.0)
    h = jnp.dot(h, w2[...], preferred_element_type=jnp.float32) + b2[...]
    hl = jnp.dot(h, cw1h[...], preferred_element_type=jnp.float32)
    out_ref[...] = dinv[:, None] * hl


def _tc_encode(x, dinv3, w1, b1, w2, b2, cw1):
    g = N // _BLK
    return pl.pallas_call(
        _enc_body,
        grid=(g, 2),
        in_specs=[
            pl.BlockSpec((_BLK, 128), lambda i, j: (i, 0)),
            pl.BlockSpec((1, 1, _BLK), lambda i, j: (i, 0, 0)),
            pl.BlockSpec((128, 256), lambda i, j: (0, 0)),
            pl.BlockSpec((1, 256), lambda i, j: (0, 0)),
            pl.BlockSpec((256, 256), lambda i, j: (0, 0)),
            pl.BlockSpec((1, 256), lambda i, j: (0, 0)),
            pl.BlockSpec((256, DH), lambda i, j: (0, j)),
        ],
        out_specs=pl.BlockSpec((_BLK, DH), lambda i, j: (j * (N // _BLK) + i, 0)),
        out_shape=jax.ShapeDtypeStruct((NC * N, DH), jnp.float32),
    )(x, dinv3, w1, b1, w2, b2, cw1)


def _mid_body(acc_lo, acc_hi, tab_lo, tab_hi, dinv_ref, b_ref, wnh_ref, out_ref):
    dinv = dinv_ref[0, 0, :]
    acc = jnp.concatenate([acc_lo[...], acc_hi[...]], axis=1)
    hlp = jnp.concatenate([tab_lo[...], tab_hi[...]], axis=1)
    t = jnp.maximum(dinv[:, None] * (acc + hlp) + b_ref[...], 0.0)
    out_ref[...] = dinv[:, None] * jnp.dot(t, wnh_ref[...],
                                           preferred_element_type=jnp.float32)


def _tc_mid(accf, table, dinv3, b, wn):
    g = N // _BLK
    return pl.pallas_call(
        _mid_body,
        grid=(g, 2),
        in_specs=[
            pl.BlockSpec((_BLK, DH), lambda i, j: (i, 0)),
            pl.BlockSpec((_BLK, DH), lambda i, j: (g + i, 0)),
            pl.BlockSpec((_BLK, DH), lambda i, j: (i, 0)),
            pl.BlockSpec((_BLK, DH), lambda i, j: (g + i, 0)),
            pl.BlockSpec((1, 1, _BLK), lambda i, j: (i, 0, 0)),
            pl.BlockSpec((1, 256), lambda i, j: (0, 0)),
            pl.BlockSpec((256, DH), lambda i, j: (0, j)),
        ],
        out_specs=pl.BlockSpec((_BLK, DH), lambda i, j: (j * g + i, 0)),
        out_shape=jax.ShapeDtypeStruct((NC * N, DH), jnp.float32),
    )(accf, accf, table, table, dinv3, b, wn)


def _final_body(acc_lo, acc_hi, tab_lo, tab_hi, dinv_ref, b3_ref,
                pw1, pb1, pw2, pb2, dw1, db1, dw2, db2,
                ge_ref, p_ref, sum_s, max_s):
    i = pl.program_id(0)
    dinv = dinv_ref[0, 0, :]
    acc = jnp.concatenate([acc_lo[...], acc_hi[...]], axis=1)
    hlp = jnp.concatenate([tab_lo[...], tab_hi[...]], axis=1)
    h3 = jnp.maximum(dinv[:, None] * (acc + hlp) + b3_ref[...], 0.0)
    bsum = jnp.sum(h3, axis=0, keepdims=True)
    bmax = jnp.max(h3, axis=0, keepdims=True)

    @pl.when(i == 0)
    def _():
        sum_s[...] = bsum
        max_s[...] = bmax

    @pl.when(i > 0)
    def _():
        sum_s[...] = sum_s[...] + bsum
        max_s[...] = jnp.maximum(max_s[...], bmax)

    @pl.when(i == N // _BLK - 1)
    def _():
        g = jnp.concatenate([sum_s[...] * (1.0 / N), max_s[...]], axis=1)
        ge = jnp.maximum(jnp.dot(g, pw1[...],
                                 preferred_element_type=jnp.float32)
                         + pb1[...], 0.0)
        ge = jnp.dot(ge, pw2[...], preferred_element_type=jnp.float32) + pb2[...]
        v = jnp.maximum(jnp.dot(ge, dw1[...],
                                preferred_element_type=jnp.float32)
                        + db1[...], 0.0)
        v = jnp.dot(v, dw2[...], preferred_element_type=jnp.float32) + db2[...]
        ge_ref[...] = ge
        p_ref[...] = jax.nn.sigmoid(jnp.sum(v * v)).reshape(1, 1)


def _tc_final(accf, table, dinv3, b3, pw1, pb1, pw2, pb2, dw1, db1, dw2, db2):
    g = N // _BLK
    hs = [pl.BlockSpec((_BLK, DH), lambda i: (i, 0)),
          pl.BlockSpec((_BLK, DH), lambda i: (g + i, 0))]
    return pl.pallas_call(
        _final_body,
        grid=(g,),
        in_specs=hs + hs + [pl.BlockSpec((1, 1, _BLK), lambda i: (i, 0, 0))]
                 + [pl.BlockSpec(x.shape, lambda i, nd=x.ndim: (0,) * nd)
                    for x in (b3, pw1, pb1, pw2, pb2, dw1, db1, dw2, db2)],
        out_specs=(pl.BlockSpec((1, 128), lambda i: (0, 0)),
                   pl.BlockSpec((1, 1), lambda i: (0, 0))),
        out_shape=(
            jax.ShapeDtypeStruct((1, 128), jnp.float32),
            jax.ShapeDtypeStruct((1, 1), jnp.float32),
        ),
        scratch_shapes=[pltpu.VMEM((1, 256), jnp.float32),
                        pltpu.VMEM((1, 256), jnp.float32)],
    )(accf, accf, table, table, dinv3, b3, pw1, pb1, pw2, pb2, dw1, db1, dw2, db2)


def kernel(x, edge_index, enc_W1, enc_b1, enc_W2, enc_b2,
           conv_W1, conv_b1, conv_W2, conv_b2, conv_W3, conv_b3,
           proj_W1, proj_b1, proj_W2, proj_b2,
           dec_W1, dec_b1, dec_W2, dec_b2):
    e = edge_index.shape[1]
    src = edge_index[0].astype(jnp.int32)
    dst = edge_index[1].astype(jnp.int32)

    ept_deg = e // (NC * NS)
    dst_deg = dst.reshape(NC * NS, ept_deg)
    degp = _deg_partials(dst_deg)

    b = 100
    nb = e // (NS * b)
    zeros = jnp.zeros((1000, DH), jnp.float32)
    src_t = src.reshape(NS, nb, b)
    src_r = jnp.concatenate([src_t, src_t + N], axis=0).reshape(NC * NS, nb, b)
    dst_r = dst.reshape(NS, nb, b)

    dinv = _tc_dinv(degp)
    dinv3 = dinv.reshape(N // _BLK, 1, _BLK)

    r2 = lambda v: v.reshape(1, -1)
    table = _tc_encode(x, dinv3, enc_W1, r2(enc_b1), enc_W2, r2(enc_b2), conv_W1)

    for (bb, wn) in ((conv_b1, conv_W2), (conv_b2, conv_W3)):
        accf = _sc_scatter(table, src_r, dst_r, zeros)
        table = _tc_mid(accf, table, dinv3, r2(bb), wn)

    accf = _sc_scatter(table, src_r, dst_r, zeros)

    ge, pv = _tc_final(accf, table, dinv3, r2(conv_b3),
                       proj_W1, r2(proj_b1), proj_W2, r2(proj_b2),
                       dec_W1, r2(dec_b1), dec_W2, r2(dec_b2))
    probs = jnp.broadcast_to(pv.reshape(()), (e,))
    return ge, probs
